# baseline (speedup 1.0000x reference)
import functools

import jax
import jax.numpy as jnp
from jax import lax
from jax.experimental import pallas as pl
from jax.experimental.pallas import tpu as pltpu
from jax.experimental.pallas import tpu_sc as plsc

NUM_LAYERS = 1000
EMBED_DIM = 512
OUT_FEATS = 64
BATCH = 16384

_CHUNK = 128


def _project_body(table_ref, w_ref, b_ref, out_ref):
    acc = jnp.dot(table_ref[...], w_ref[...], preferred_element_type=jnp.float32)
    out_ref[...] = jnp.maximum(acc + b_ref[...], 0.0)


def _project(emb_table, W, b):
    return pl.pallas_call(
        _project_body,
        out_shape=jax.ShapeDtypeStruct((NUM_LAYERS, OUT_FEATS), jnp.float32),
    )(emb_table, W, b.reshape(1, OUT_FEATS))


@functools.cache
def _make_gather(num_cores, num_subcores):
    nw = num_cores * num_subcores
    b_per_w = BATCH // nw
    chunks = b_per_w // _CHUNK
    mesh = plsc.VectorSubcoreMesh(core_axis_name="c", subcore_axis_name="s")

    @functools.partial(
        pl.kernel,
        mesh=mesh,
        compiler_params=pltpu.CompilerParams(use_tc_tiling_on_sc=False),
        out_type=jax.ShapeDtypeStruct((BATCH, OUT_FEATS), jnp.float32),
        scratch_types=[
            pltpu.VMEM((chunks, _CHUNK), jnp.int32),
            pltpu.VMEM((b_per_w, OUT_FEATS), jnp.float32),
            pltpu.SemaphoreType.DMA,
        ],
    )
    def gather(table_hbm, idx_hbm, out_hbm, idx_v, rows_v, sem):
        wid = lax.axis_index("s") * num_cores + lax.axis_index("c")
        base = wid * b_per_w
        pltpu.sync_copy(idx_hbm.at[pl.ds(wid * chunks, chunks)], idx_v)
        copies = []
        for j in range(chunks):
            copies.append(
                pltpu.async_copy(
                    table_hbm.at[idx_v.at[j]],
                    rows_v.at[pl.ds(j * _CHUNK, _CHUNK)],
                    sem,
                )
            )
        for c in copies:
            c.wait()
        pltpu.sync_copy(rows_v, out_hbm.at[pl.ds(base, b_per_w)])

    return gather


def kernel(layer_idx, emb_table, W, b):
    proj = _project(emb_table, W, b)
    out = jnp.take(proj, layer_idx, axis=0)
    return out.reshape(BATCH, 1, 8, 8)


# speedup vs baseline: 1.3482x; 1.3482x over previous
import functools

import jax
import jax.numpy as jnp
from jax import lax
from jax.experimental import pallas as pl
from jax.experimental.pallas import tpu as pltpu
from jax.experimental.pallas import tpu_sc as plsc

NUM_LAYERS = 1000
EMBED_DIM = 512
OUT_FEATS = 64
BATCH = 16384

_CHUNK = 128
_L = 16


def _project_body(table_ref, w_ref, b_ref, out_ref):
    acc = jnp.dot(table_ref[...], w_ref[...], preferred_element_type=jnp.float32)
    out_ref[...] = jnp.maximum(acc + b_ref[...], 0.0)


def _project(emb_table, W, b):
    return pl.pallas_call(
        _project_body,
        out_shape=jax.ShapeDtypeStruct((NUM_LAYERS, OUT_FEATS), jnp.float32),
    )(emb_table, W, b.reshape(1, OUT_FEATS))


@functools.cache
def _make_gather(num_cores, num_subcores):
    nw = num_cores * num_subcores
    b_per_w = BATCH // nw
    chunks = b_per_w // _CHUNK
    groups = b_per_w // _L
    mesh = plsc.VectorSubcoreMesh(core_axis_name="c", subcore_axis_name="s")

    @functools.partial(
        pl.kernel,
        mesh=mesh,
        compiler_params=pltpu.CompilerParams(
            use_tc_tiling_on_sc=False, needs_layout_passes=False
        ),
        out_type=jax.ShapeDtypeStruct((OUT_FEATS, BATCH), jnp.float32),
        scratch_types=[
            pltpu.VMEM((chunks, _CHUNK), jnp.int32),
            pltpu.VMEM((b_per_w, OUT_FEATS), jnp.float32),
            pltpu.VMEM((OUT_FEATS, b_per_w), jnp.float32),
            pltpu.SemaphoreType.DMA,
        ],
    )
    def gather(table_hbm, idx_hbm, out_hbm, idx_v, rows_v, rows_t_v, sem):
        wid = lax.axis_index("s") * num_cores + lax.axis_index("c")
        base = wid * b_per_w
        pltpu.sync_copy(idx_hbm.at[pl.ds(wid * chunks, chunks)], idx_v)
        copies = []
        for j in range(chunks):
            copies.append(
                pltpu.async_copy(
                    table_hbm.at[idx_v.at[j]],
                    rows_v.at[pl.ds(j * _CHUNK, _CHUNK)],
                    sem,
                )
            )
        for c in copies:
            c.wait()
        lanes = lax.iota(jnp.int32, _L)

        @plsc.parallel_loop(0, groups, 1, unroll=2)
        def _(g):
            row_ids = g * _L + lanes
            for f in range(OUT_FEATS):
                col_ids = jnp.full((_L,), f, jnp.int32)
                vals = plsc.load_gather(rows_v, [row_ids, col_ids])
                rows_t_v[f, pl.ds(g * _L, _L)] = vals

        pltpu.sync_copy(rows_t_v, out_hbm.at[:, pl.ds(base, b_per_w)])

    return gather


def kernel(layer_idx, emb_table, W, b):
    proj = _project(emb_table, W, b)
    info = plsc.get_sparse_core_info()
    gather = _make_gather(info.num_cores, info.num_subcores)
    idx2d = layer_idx.astype(jnp.int32).reshape(BATCH // _CHUNK, _CHUNK)
    out_t = gather(proj, idx2d)
    return out_t.reshape(1, 8, 8, BATCH).transpose(3, 0, 1, 2)


# speedup vs baseline: 1.4522x; 1.0771x over previous
import functools

import jax
import jax.numpy as jnp
from jax import lax
from jax.experimental import pallas as pl
from jax.experimental.pallas import tpu as pltpu
from jax.experimental.pallas import tpu_sc as plsc

NUM_LAYERS = 1000
EMBED_DIM = 512
OUT_FEATS = 64
BATCH = 16384

_CHUNK = 128
_L = 16


def _project_body(table_ref, w_ref, b_ref, out_ref):
    acc = jnp.dot(table_ref[...], w_ref[...], preferred_element_type=jnp.float32)
    out_ref[...] = jnp.maximum(acc + b_ref[...], 0.0)


def _project(emb_table, W, b):
    return pl.pallas_call(
        _project_body,
        out_shape=jax.ShapeDtypeStruct((NUM_LAYERS, OUT_FEATS), jnp.float32),
    )(emb_table, W, b.reshape(1, OUT_FEATS))


@functools.cache
def _make_gather(num_cores, num_subcores):
    nw = num_cores * num_subcores
    b_per_w = BATCH // nw
    chunks = b_per_w // _CHUNK
    groups = b_per_w // _L
    mesh = plsc.VectorSubcoreMesh(core_axis_name="c", subcore_axis_name="s")

    @functools.partial(
        pl.kernel,
        mesh=mesh,
        compiler_params=pltpu.CompilerParams(
            use_tc_tiling_on_sc=False, needs_layout_passes=False
        ),
        out_type=jax.ShapeDtypeStruct((OUT_FEATS, BATCH), jnp.float32),
        scratch_types=[
            pltpu.VMEM((chunks, _CHUNK), jnp.int32),
            pltpu.VMEM((b_per_w, OUT_FEATS), jnp.float32),
            pltpu.VMEM((OUT_FEATS, b_per_w), jnp.float32),
            pltpu.SemaphoreType.DMA,
        ],
    )
    def gather(table_hbm, idx_hbm, out_hbm, idx_v, rows_v, rows_t_v, sem):
        wid = lax.axis_index("s") * num_cores + lax.axis_index("c")
        base = wid * b_per_w
        pltpu.sync_copy(idx_hbm.at[pl.ds(wid * chunks, chunks)], idx_v)
        copies = []
        for j in range(chunks):
            copies.append(
                pltpu.async_copy(
                    table_hbm.at[idx_v.at[j]],
                    rows_v.at[pl.ds(j * _CHUNK, _CHUNK)],
                    sem,
                )
            )
        for c in copies:
            c.wait()
        lanes = lax.iota(jnp.int32, _L)

        @plsc.parallel_loop(0, OUT_FEATS, 1)
        def _(f):
            col_ids = jnp.full((_L,), 0, jnp.int32) + f
            for g in range(groups):
                row_ids = g * _L + lanes
                vals = plsc.load_gather(rows_v, [row_ids, col_ids])
                rows_t_v[f, pl.ds(g * _L, _L)] = vals

        pltpu.sync_copy(rows_t_v, out_hbm.at[:, pl.ds(base, b_per_w)])

    return gather


def kernel(layer_idx, emb_table, W, b):
    proj = _project(emb_table, W, b)
    info = plsc.get_sparse_core_info()
    gather = _make_gather(info.num_cores, info.num_subcores)
    idx2d = layer_idx.astype(jnp.int32).reshape(BATCH // _CHUNK, _CHUNK)
    out_t = gather(proj, idx2d)
    return out_t.reshape(1, 8, 8, BATCH).transpose(3, 0, 1, 2)


# speedup vs baseline: 1.9173x; 1.3203x over previous
import functools

import jax
import jax.numpy as jnp
from jax import lax
from jax.experimental import pallas as pl
from jax.experimental.pallas import tpu as pltpu
from jax.experimental.pallas import tpu_sc as plsc

NUM_LAYERS = 1000
EMBED_DIM = 512
OUT_FEATS = 64
BATCH = 16384

_CHUNK = 128
_L = 16


def _project_body(table_ref, w_ref, b_ref, out_ref):
    acc = jnp.dot(table_ref[...], w_ref[...], preferred_element_type=jnp.float32)
    out_ref[...] = jnp.maximum(acc + b_ref[...], 0.0)


def _project(emb_table, W, b):
    return pl.pallas_call(
        _project_body,
        out_shape=jax.ShapeDtypeStruct((NUM_LAYERS, OUT_FEATS), jnp.float32),
    )(emb_table, W, b.reshape(1, OUT_FEATS))


@functools.cache
def _make_gather(num_cores, num_subcores):
    nw = num_cores * num_subcores
    b_per_w = BATCH // nw
    chunks = b_per_w // _CHUNK
    groups = b_per_w // _L
    mesh = plsc.VectorSubcoreMesh(core_axis_name="c", subcore_axis_name="s")

    @functools.partial(
        pl.kernel,
        mesh=mesh,
        compiler_params=pltpu.CompilerParams(
            use_tc_tiling_on_sc=False, needs_layout_passes=False
        ),
        out_type=jax.ShapeDtypeStruct((OUT_FEATS, BATCH), jnp.float32),
        scratch_types=[
            pltpu.VMEM((chunks, _CHUNK), jnp.int32),
            pltpu.VMEM((b_per_w, OUT_FEATS), jnp.float32),
            pltpu.VMEM((OUT_FEATS, b_per_w + 1), jnp.float32),
            pltpu.SemaphoreType.DMA,
        ],
    )
    def gather(table_hbm, idx_hbm, out_hbm, idx_v, rows_v, rows_t_v, sem):
        wid = lax.axis_index("s") * num_cores + lax.axis_index("c")
        base = wid * b_per_w
        pltpu.sync_copy(idx_hbm.at[pl.ds(wid * chunks, chunks)], idx_v)
        copies = []
        for j in range(chunks):
            copies.append(
                pltpu.async_copy(
                    table_hbm.at[idx_v.at[j]],
                    rows_v.at[pl.ds(j * _CHUNK, _CHUNK)],
                    sem,
                )
            )
        for c in copies:
            c.wait()
        lanes = lax.iota(jnp.int32, _L)

        @plsc.parallel_loop(0, b_per_w, 1)
        def _(b):
            col = jnp.full((_L,), 0, jnp.int32) + b
            for q in range(OUT_FEATS // _L):
                vals = rows_v[b, pl.ds(q * _L, _L)]
                plsc.store_scatter(rows_t_v, [q * _L + lanes, col], vals)

        pltpu.sync_copy(
            rows_t_v.at[:, pl.ds(0, b_per_w)], out_hbm.at[:, pl.ds(base, b_per_w)]
        )

    return gather


def kernel(layer_idx, emb_table, W, b):
    proj = _project(emb_table, W, b)
    info = plsc.get_sparse_core_info()
    gather = _make_gather(info.num_cores, info.num_subcores)
    idx2d = layer_idx.astype(jnp.int32).reshape(BATCH // _CHUNK, _CHUNK)
    out_t = gather(proj, idx2d)
    return out_t.reshape(1, 8, 8, BATCH).transpose(3, 0, 1, 2)


# speedup vs baseline: 1.9193x; 1.0011x over previous
import functools

import jax
import jax.numpy as jnp
from jax import lax
from jax.experimental import pallas as pl
from jax.experimental.pallas import tpu as pltpu
from jax.experimental.pallas import tpu_sc as plsc

NUM_LAYERS = 1000
EMBED_DIM = 512
OUT_FEATS = 64
BATCH = 16384

_CHUNK = 128
_L = 16


def _project_body(table_ref, w_ref, b_ref, out_ref):
    acc = jnp.dot(table_ref[...], w_ref[...], preferred_element_type=jnp.float32)
    out_ref[...] = jnp.maximum(acc + b_ref[...], 0.0)


def _project(emb_table, W, b):
    return pl.pallas_call(
        _project_body,
        out_shape=jax.ShapeDtypeStruct((NUM_LAYERS, OUT_FEATS), jnp.float32),
    )(emb_table, W, b.reshape(1, OUT_FEATS))


@functools.cache
def _make_gather(num_cores, num_subcores):
    nw = num_cores * num_subcores
    b_per_w = BATCH // nw
    chunks = b_per_w // _CHUNK
    groups = b_per_w // _L
    mesh = plsc.VectorSubcoreMesh(core_axis_name="c", subcore_axis_name="s")

    @functools.partial(
        pl.kernel,
        mesh=mesh,
        compiler_params=pltpu.CompilerParams(
            use_tc_tiling_on_sc=False,
            needs_layout_passes=False,
            skip_device_barrier=True,
        ),
        out_type=jax.ShapeDtypeStruct((OUT_FEATS, BATCH), jnp.float32),
        scratch_types=[
            pltpu.VMEM((chunks, _CHUNK), jnp.int32),
            pltpu.VMEM((b_per_w, OUT_FEATS), jnp.float32),
            pltpu.VMEM((OUT_FEATS, b_per_w + 1), jnp.float32),
            pltpu.SemaphoreType.DMA,
        ],
    )
    def gather(table_hbm, idx_hbm, out_hbm, idx_v, rows_v, rows_t_v, sem):
        wid = lax.axis_index("s") * num_cores + lax.axis_index("c")
        base = wid * b_per_w
        pltpu.sync_copy(idx_hbm.at[pl.ds(wid * chunks, chunks)], idx_v)
        copies = []
        for j in range(chunks):
            copies.append(
                pltpu.async_copy(
                    table_hbm.at[idx_v.at[j]],
                    rows_v.at[pl.ds(j * _CHUNK, _CHUNK)],
                    sem,
                )
            )
        for c in copies:
            c.wait()
        lanes = lax.iota(jnp.int32, _L)

        @plsc.parallel_loop(0, b_per_w, 1)
        def _(b):
            col = jnp.full((_L,), 0, jnp.int32) + b
            for q in range(OUT_FEATS // _L):
                vals = rows_v[b, pl.ds(q * _L, _L)]
                plsc.store_scatter(rows_t_v, [q * _L + lanes, col], vals)

        pltpu.sync_copy(
            rows_t_v.at[:, pl.ds(0, b_per_w)], out_hbm.at[:, pl.ds(base, b_per_w)]
        )

    return gather


def kernel(layer_idx, emb_table, W, b):
    proj = _project(emb_table, W, b)
    info = plsc.get_sparse_core_info()
    gather = _make_gather(info.num_cores, info.num_subcores)
    idx2d = layer_idx.astype(jnp.int32).reshape(BATCH // _CHUNK, _CHUNK)
    out_t = gather(proj, idx2d)
    return out_t.reshape(1, 8, 8, BATCH).transpose(3, 0, 1, 2)


# speedup vs baseline: 2.2357x; 1.1648x over previous
import functools

import jax
import jax.numpy as jnp
from jax import lax
from jax.experimental import pallas as pl
from jax.experimental.pallas import tpu as pltpu
from jax.experimental.pallas import tpu_sc as plsc

NUM_LAYERS = 1000
EMBED_DIM = 512
OUT_FEATS = 64
BATCH = 16384

_CHUNK = 128
_L = 16


def _project_body(table_ref, w_ref, b_ref, out_ref):
    acc = jnp.dot(table_ref[...], w_ref[...], preferred_element_type=jnp.float32)
    out_ref[...] = jnp.maximum(acc + b_ref[...], 0.0)


def _project(emb_table, W, b):
    return pl.pallas_call(
        _project_body,
        out_shape=jax.ShapeDtypeStruct((NUM_LAYERS, OUT_FEATS), jnp.float32),
    )(emb_table, W, b.reshape(1, OUT_FEATS))


@functools.cache
def _make_gather(num_cores, num_subcores):
    nw = num_cores * num_subcores
    b_per_w = BATCH // nw
    chunks = b_per_w // _CHUNK
    groups = b_per_w // _L
    mesh = plsc.VectorSubcoreMesh(core_axis_name="c", subcore_axis_name="s")

    @functools.partial(
        pl.kernel,
        mesh=mesh,
        compiler_params=pltpu.CompilerParams(
            use_tc_tiling_on_sc=False,
            needs_layout_passes=False,
            skip_device_barrier=True,
        ),
        out_type=jax.ShapeDtypeStruct((8, BATCH // _CHUNK, 8, _CHUNK), jnp.float32),
        scratch_types=[
            pltpu.VMEM((chunks, _CHUNK), jnp.int32),
            pltpu.VMEM((b_per_w, OUT_FEATS), jnp.float32),
            pltpu.VMEM((8, 8, b_per_w + 1), jnp.float32),
            pltpu.SemaphoreType.DMA,
        ],
    )
    def gather(table_hbm, idx_hbm, out_hbm, idx_v, rows_v, rows_t_v, sem):
        wid = lax.axis_index("s") * num_cores + lax.axis_index("c")
        base = wid * b_per_w
        pltpu.sync_copy(idx_hbm.at[pl.ds(wid * chunks, chunks)], idx_v)
        copies = []
        for j in range(chunks):
            copies.append(
                pltpu.async_copy(
                    table_hbm.at[idx_v.at[j]],
                    rows_v.at[pl.ds(j * _CHUNK, _CHUNK)],
                    sem,
                )
            )
        for c in copies:
            c.wait()
        lanes = lax.iota(jnp.int32, _L)
        i_ids = [(q * _L + lanes) // 8 for q in range(OUT_FEATS // _L)]
        j_ids = [(q * _L + lanes) % 8 for q in range(OUT_FEATS // _L)]

        @plsc.parallel_loop(0, b_per_w, 1)
        def _(b):
            col = jnp.full((_L,), 0, jnp.int32) + b
            for q in range(OUT_FEATS // _L):
                vals = rows_v[b, pl.ds(q * _L, _L)]
                plsc.store_scatter(rows_t_v, [i_ids[q], j_ids[q], col], vals)

        for t in range(chunks):
            pltpu.sync_copy(
                rows_t_v.at[:, :, pl.ds(t * _CHUNK, _CHUNK)],
                out_hbm.at[:, wid * chunks + t, :, :],
            )

    return gather


def kernel(layer_idx, emb_table, W, b):
    proj = _project(emb_table, W, b)
    info = plsc.get_sparse_core_info()
    gather = _make_gather(info.num_cores, info.num_subcores)
    idx2d = layer_idx.astype(jnp.int32).reshape(BATCH // _CHUNK, _CHUNK)
    out4 = gather(proj, idx2d)
    return out4.transpose(1, 3, 0, 2).reshape(BATCH, 1, 8, 8)
